# baseline (device time: 50942 ns/iter reference)
import jax
import jax.numpy as jnp
from jax import lax
from jax.experimental import pallas as pl
from jax.experimental.pallas import tpu as pltpu

N_DEV = 8
SUB = 4


def kernel(x):
    _, m, n = x.shape
    ch = n // N_DEV
    half = m // 2
    sub = half // SUB

    def body(x_ref, out_ref, comm_r, comm_l,
             send_r, recv_r, send_l, recv_l):
        my = lax.axis_index("i")
        left = (my + N_DEV - 1) % N_DEV
        right = (my + 1) % N_DEV

        barrier_sem = pltpu.get_barrier_semaphore()
        for nbr in (left, right):
            pl.semaphore_signal(
                barrier_sem, inc=1,
                device_id=(nbr,), device_id_type=pl.DeviceIdType.MESH,
            )
        pl.semaphore_wait(barrier_sem, 2)

        def rdma(dir_comm, dir_send, dir_recv, s, t, dst):
            return pltpu.make_async_remote_copy(
                src_ref=dir_comm.at[s, t],
                dst_ref=dir_comm.at[s + 1, t],
                send_sem=dir_send.at[s, t],
                recv_sem=dir_recv.at[s, t],
                device_id=(dst,),
                device_id_type=pl.DeviceIdType.MESH,
            )

        def row(t):
            return pl.ds(t * sub, sub)

        def c_right(s):
            return (my + 2 * N_DEV - 2 - s) % N_DEV

        def c_left(s):
            return (my + 2 + s) % N_DEV

        for t in range(SUB):
            comm_r[0, t, :, :] = x_ref[0, t * sub:(t + 1) * sub,
                                       pl.ds(c_right(-1) * ch, ch)]
            comm_l[0, t, :, :] = x_ref[0, half + t * sub:half + (t + 1) * sub,
                                       pl.ds(c_left(-1) * ch, ch)]
            rdma(comm_r, send_r, recv_r, 0, t, right).start()
            rdma(comm_l, send_l, recv_l, 0, t, left).start()

        for s in range(N_DEV - 1):
            last = s == N_DEV - 2
            for t in range(SUB):
                r = rdma(comm_r, send_r, recv_r, s, t, right)
                l = rdma(comm_l, send_l, recv_l, s, t, left)
                r.wait_recv()
                top = x_ref[0, t * sub:(t + 1) * sub,
                            pl.ds(c_right(s) * ch, ch)]
                if last:
                    out_ref[row(t), :] = comm_r[s + 1, t, :, :] + top
                else:
                    comm_r[s + 1, t, :, :] = comm_r[s + 1, t, :, :] + top
                    rdma(comm_r, send_r, recv_r, s + 1, t, right).start()
                l.wait_recv()
                bot = x_ref[0, half + t * sub:half + (t + 1) * sub,
                            pl.ds(c_left(s) * ch, ch)]
                if last:
                    out_ref[pl.ds(half + t * sub, sub), :] = (
                        comm_l[s + 1, t, :, :] + bot
                    )
                else:
                    comm_l[s + 1, t, :, :] = comm_l[s + 1, t, :, :] + bot
                    rdma(comm_l, send_l, recv_l, s + 1, t, left).start()

        for s in range(N_DEV - 1):
            for t in range(SUB):
                rdma(comm_r, send_r, recv_r, s, t, right).wait_send()
                rdma(comm_l, send_l, recv_l, s, t, left).wait_send()

    return pl.pallas_call(
        body,
        out_shape=jax.ShapeDtypeStruct((m, ch), jnp.float32),
        in_specs=[pl.BlockSpec(memory_space=pltpu.VMEM)],
        out_specs=pl.BlockSpec(memory_space=pltpu.VMEM),
        scratch_shapes=[
            pltpu.VMEM((N_DEV, SUB, sub, ch), jnp.float32),
            pltpu.VMEM((N_DEV, SUB, sub, ch), jnp.float32),
            pltpu.SemaphoreType.DMA((N_DEV - 1, SUB)),
            pltpu.SemaphoreType.DMA((N_DEV - 1, SUB)),
            pltpu.SemaphoreType.DMA((N_DEV - 1, SUB)),
            pltpu.SemaphoreType.DMA((N_DEV - 1, SUB)),
        ],
        compiler_params=pltpu.CompilerParams(collective_id=0),
    )(x)


# device time: 40536 ns/iter; 1.2567x vs baseline; 1.2567x over previous
import jax
import jax.numpy as jnp
from jax import lax
from jax.experimental import pallas as pl
from jax.experimental.pallas import tpu as pltpu

N_DEV = 8

PARTS = (
    (0, 176, (1, 3, 4)),
    (176, 168, (3, 4, 1)),
    (344, 168, (4, 1, 3)),
)

_CS = ((0, 0), (0, 1), (1, 0), (1, 1))


def kernel(x):
    _, m, n = x.shape
    ch = n // N_DEV

    scratch = []
    for _, rp, _ in PARTS:
        scratch += [
            pltpu.VMEM((4, rp, ch), jnp.float32),
            pltpu.VMEM((4, rp, ch), jnp.float32),
            pltpu.VMEM((2, rp, ch), jnp.float32),
            pltpu.VMEM((2, rp, ch), jnp.float32),
            pltpu.VMEM((rp, ch), jnp.float32),
            pltpu.SemaphoreType.DMA((4,)),
            pltpu.SemaphoreType.DMA((4,)),
            pltpu.SemaphoreType.DMA,
            pltpu.SemaphoreType.DMA,
            pltpu.SemaphoreType.DMA,
            pltpu.SemaphoreType.DMA,
        ]

    def body(x_ref, out_ref, *sc):
        my = lax.axis_index("i")
        parts = [sc[11 * p:11 * (p + 1)] for p in range(3)]

        barrier_sem = pltpu.get_barrier_semaphore()
        for mask in (1, 3, 4):
            pl.semaphore_signal(
                barrier_sem, inc=1,
                device_id=(my ^ mask,), device_id_type=pl.DeviceIdType.MESH,
            )
        pl.semaphore_wait(barrier_sem, 3)

        drain = []

        r1 = []
        for p, (row0, rp, (m1, m2, m3)) in enumerate(PARTS):
            R1, _, _, _, _, s1s, s1r = parts[p][:7]
            rds = []
            for idx, (c2, c3) in enumerate(_CS):
                j = my ^ (m1 ^ (m2 if c2 else 0) ^ (m3 if c3 else 0))
                r = pltpu.make_async_remote_copy(
                    src_ref=x_ref.at[0, pl.ds(row0, rp), pl.ds(j * ch, ch)],
                    dst_ref=R1.at[idx],
                    send_sem=s1s.at[idx],
                    recv_sem=s1r.at[idx],
                    device_id=(my ^ m1,),
                    device_id_type=pl.DeviceIdType.MESH,
                )
                r.start()
                rds.append(r)
            r1.append(rds)
            drain += rds

        r2 = []
        for p, (row0, rp, (m1, m2, m3)) in enumerate(PARTS):
            R1, A2, R2, _, _, _, _, s2s, s2r = parts[p][:9]
            for idx in (2, 3, 0, 1):
                c2, c3 = _CS[idx]
                r1[p][idx].wait_recv()
                j = my ^ ((m2 if c2 else 0) ^ (m3 if c3 else 0))
                A2[idx, :, :] = (
                    x_ref[0, pl.ds(row0, rp), pl.ds(j * ch, ch)]
                    + R1[idx, :, :]
                )
                if idx == 3:
                    r = pltpu.make_async_remote_copy(
                        src_ref=A2.at[pl.ds(2, 2)],
                        dst_ref=R2,
                        send_sem=s2s,
                        recv_sem=s2r,
                        device_id=(my ^ m2,),
                        device_id_type=pl.DeviceIdType.MESH,
                    )
                    r.start()
                    r2.append(r)
                    drain.append(r)

        r3 = []
        for p, (row0, rp, (m1, m2, m3)) in enumerate(PARTS):
            _, A2, R2, A3, _, _, _, _, _, s3s, s3r = parts[p]
            r2[p].wait_recv()
            A3[1, :, :] = A2[1, :, :] + R2[1, :, :]
            r = pltpu.make_async_remote_copy(
                src_ref=A3.at[1],
                dst_ref=parts[p][4],
                send_sem=s3s,
                recv_sem=s3r,
                device_id=(my ^ m3,),
                device_id_type=pl.DeviceIdType.MESH,
            )
            r.start()
            r3.append(r)
            drain.append(r)
            A3[0, :, :] = A2[0, :, :] + R2[0, :, :]

        for p, (row0, rp, _) in enumerate(PARTS):
            A3, R3 = parts[p][3], parts[p][4]
            r3[p].wait_recv()
            out_ref[pl.ds(row0, rp), :] = A3[0, :, :] + R3[:, :]

        for r in drain:
            r.wait_send()

    return pl.pallas_call(
        body,
        out_shape=jax.ShapeDtypeStruct((m, ch), jnp.float32),
        in_specs=[pl.BlockSpec(memory_space=pltpu.VMEM)],
        out_specs=pl.BlockSpec(memory_space=pltpu.VMEM),
        scratch_shapes=scratch,
        compiler_params=pltpu.CompilerParams(collective_id=0),
    )(x)


# device time: 36582 ns/iter; 1.3925x vs baseline; 1.1081x over previous
import jax
import jax.numpy as jnp
from jax import lax
from jax.experimental import pallas as pl
from jax.experimental.pallas import tpu as pltpu

N_DEV = 8

PARTS = (
    (0, 176, (1, 3, 4)),
    (176, 168, (3, 4, 1)),
    (344, 168, (4, 1, 3)),
)

_CS = ((0, 0), (0, 1), (1, 0), (1, 1))


def kernel(x):
    _, m, n = x.shape
    ch = n // N_DEV

    scratch = []
    for _, rp, _ in PARTS:
        scratch += [
            pltpu.VMEM((4, rp, ch), jnp.float32),
            pltpu.VMEM((4, rp, ch), jnp.float32),
            pltpu.VMEM((2, rp, ch), jnp.float32),
            pltpu.VMEM((2, rp, ch), jnp.float32),
            pltpu.VMEM((rp, ch), jnp.float32),
            pltpu.SemaphoreType.DMA((4,)),
            pltpu.SemaphoreType.DMA((4,)),
            pltpu.SemaphoreType.DMA((2,)),
            pltpu.SemaphoreType.DMA((2,)),
            pltpu.SemaphoreType.DMA,
            pltpu.SemaphoreType.DMA,
        ]

    def body(x_ref, out_ref, *sc):
        my = lax.axis_index("i")
        parts = [sc[11 * p:11 * (p + 1)] for p in range(3)]

        barrier_sem = pltpu.get_barrier_semaphore()
        for mask in (1, 3, 4):
            pl.semaphore_signal(
                barrier_sem, inc=1,
                device_id=(my ^ mask,), device_id_type=pl.DeviceIdType.MESH,
            )
        pl.semaphore_wait(barrier_sem, 3)

        drain = []

        def xchunk(row0, rp, j):
            return x_ref[0, pl.ds(row0, rp), pl.ds(j * ch, ch)]

        r1 = [[None] * 4 for _ in range(3)]
        for p, (row0, rp, (m1, m2, m3)) in enumerate(PARTS):
            R1, _, _, _, _, s1s, s1r = parts[p][:7]
            for idx in (2, 3, 0, 1):
                c2, c3 = _CS[idx]
                j = my ^ (m1 ^ (m2 if c2 else 0) ^ (m3 if c3 else 0))
                r = pltpu.make_async_remote_copy(
                    src_ref=x_ref.at[0, pl.ds(row0, rp), pl.ds(j * ch, ch)],
                    dst_ref=R1.at[idx],
                    send_sem=s1s.at[idx],
                    recv_sem=s1r.at[idx],
                    device_id=(my ^ m1,),
                    device_id_type=pl.DeviceIdType.MESH,
                )
                r.start()
                r1[p][idx] = r
                drain.append(r)

        r2 = [[None, None] for _ in range(3)]
        for p, (row0, rp, (m1, m2, m3)) in enumerate(PARTS):
            R1, A2, R2, _, _, _, _, s2s, s2r = parts[p][:9]
            for idx in (3, 2):
                c2, c3 = _CS[idx]
                r1[p][idx].wait_recv()
                j = my ^ ((m2 if c2 else 0) ^ (m3 if c3 else 0))
                A2[idx, :, :] = xchunk(row0, rp, j) + R1[idx, :, :]
                r = pltpu.make_async_remote_copy(
                    src_ref=A2.at[idx],
                    dst_ref=R2.at[idx - 2],
                    send_sem=s2s.at[idx - 2],
                    recv_sem=s2r.at[idx - 2],
                    device_id=(my ^ m2,),
                    device_id_type=pl.DeviceIdType.MESH,
                )
                r.start()
                r2[p][idx - 2] = r
                drain.append(r)
        for p, (row0, rp, (m1, m2, m3)) in enumerate(PARTS):
            R1, A2 = parts[p][:2]
            for idx in (1, 0):
                c2, c3 = _CS[idx]
                r1[p][idx].wait_recv()
                j = my ^ ((m2 if c2 else 0) ^ (m3 if c3 else 0))
                A2[idx, :, :] = xchunk(row0, rp, j) + R1[idx, :, :]

        r3 = [None] * 3
        for p, (row0, rp, (m1, m2, m3)) in enumerate(PARTS):
            _, A2, R2, A3, R3 = parts[p][:5]
            s3s, s3r = parts[p][9], parts[p][10]
            r2[p][1].wait_recv()
            A3[1, :, :] = A2[1, :, :] + R2[1, :, :]
            r = pltpu.make_async_remote_copy(
                src_ref=A3.at[1],
                dst_ref=R3,
                send_sem=s3s,
                recv_sem=s3r,
                device_id=(my ^ m3,),
                device_id_type=pl.DeviceIdType.MESH,
            )
            r.start()
            r3[p] = r
            drain.append(r)
        for p in range(3):
            _, A2, R2, A3 = parts[p][:4]
            r2[p][0].wait_recv()
            A3[0, :, :] = A2[0, :, :] + R2[0, :, :]

        for p, (row0, rp, _) in enumerate(PARTS):
            A3, R3 = parts[p][3], parts[p][4]
            r3[p].wait_recv()
            out_ref[pl.ds(row0, rp), :] = A3[0, :, :] + R3[:, :]

        for r in drain:
            r.wait_send()

    return pl.pallas_call(
        body,
        out_shape=jax.ShapeDtypeStruct((m, ch), jnp.float32),
        in_specs=[pl.BlockSpec(memory_space=pltpu.VMEM)],
        out_specs=pl.BlockSpec(memory_space=pltpu.VMEM),
        scratch_shapes=scratch,
        compiler_params=pltpu.CompilerParams(collective_id=0),
    )(x)
